# Initial kernel scaffold; baseline (speedup 1.0000x reference)
import jax
import jax.numpy as jnp
from jax.experimental import pallas as pl


def kernel(x, table):
    raise NotImplementedError("write your pallas kernel here")


# speedup vs baseline: 1.0621x; 1.0621x over previous
import functools
import math

import jax
import jax.numpy as jnp
import numpy as np
from jax import lax
from jax.experimental import pallas as pl
from jax.experimental.pallas import tpu as pltpu
from jax.experimental.pallas import tpu_sc as plsc

D = 128
B = 4
L = 2048
BL = B * L
SCALE = math.sqrt(float(D))

_NC = 2
_NS = 16
_NW = _NC * _NS
_BPW = BL // _NW
_LANES = 16


def _pos_encoding(length, depth):
    half = depth // 2
    positions = np.arange(length)[:, None]
    depths = np.arange(half)[None, :] / half
    angle_rates = 1 / 10000 ** depths
    angle_rads = positions * angle_rates
    enc = np.concatenate([np.sin(angle_rads), np.cos(angle_rads)], axis=-1)
    return enc.astype(np.float32)


_PE = _pos_encoding(L, D)


def _make_sc_embed():
    mesh = plsc.VectorSubcoreMesh(
        core_axis_name="c", subcore_axis_name="s", num_cores=_NC
    )

    @functools.partial(
        pl.kernel,
        mesh=mesh,
        out_type=jax.ShapeDtypeStruct((BL, D), jnp.float32),
        scratch_types=[
            pltpu.VMEM((_BPW,), jnp.int32),
            pltpu.VMEM((_BPW, D), jnp.float32),
            pltpu.VMEM((_BPW, D), jnp.float32),
            pltpu.SemaphoreType.DMA,
        ],
    )
    def sc_embed(idx_hbm, table_hbm, pe_hbm, out_hbm, idx_v, rows_v, pe_v, sem):
        wid = lax.axis_index("s") * _NC + lax.axis_index("c")
        base = wid * _BPW
        pltpu.sync_copy(idx_hbm.at[pl.ds(base, _BPW)], idx_v)
        gather = pltpu.async_copy(table_hbm.at[idx_v], rows_v, sem)
        pltpu.sync_copy(pe_hbm.at[pl.ds(base % L, _BPW)], pe_v)
        gather.wait()

        def body(r, carry):
            for j in range(D // _LANES):
                sl = pl.ds(j * _LANES, _LANES)
                rows_v[r, sl] = rows_v[r, sl] * SCALE + pe_v[r, sl]
            return carry

        lax.fori_loop(0, _BPW, body, 0)
        pltpu.sync_copy(rows_v, out_hbm.at[pl.ds(base, _BPW)])

    return sc_embed


_sc_embed = _make_sc_embed()


def kernel(x, table):
    idx = x.reshape(-1)
    pe = jnp.asarray(_PE)
    out = _sc_embed(idx, table, pe)
    return out.reshape(B, L, D)


# speedup vs baseline: 1.1733x; 1.1046x over previous
import functools
import math

import jax
import jax.numpy as jnp
import numpy as np
from jax import lax
from jax.experimental import pallas as pl
from jax.experimental.pallas import tpu as pltpu
from jax.experimental.pallas import tpu_sc as plsc

D = 128
B = 4
L = 2048
BL = B * L
SCALE = math.sqrt(float(D))

_NC = 2
_NS = 16
_NW = _NC * _NS
_PPW = L // _NW
_LANES = 16


def _pos_encoding(length, depth):
    half = depth // 2
    positions = np.arange(length)[:, None]
    depths = np.arange(half)[None, :] / half
    angle_rates = 1 / 10000 ** depths
    angle_rads = positions * angle_rates
    enc = np.concatenate([np.sin(angle_rads), np.cos(angle_rads)], axis=-1)
    return enc.astype(np.float32)


_PE = _pos_encoding(L, D)


def _make_sc_embed():
    mesh = plsc.VectorSubcoreMesh(
        core_axis_name="c", subcore_axis_name="s", num_cores=_NC
    )

    @functools.partial(
        pl.kernel,
        mesh=mesh,
        out_type=jax.ShapeDtypeStruct((BL, D), jnp.float32),
        scratch_types=[
            pltpu.VMEM((B, _PPW), jnp.int32),
            pltpu.VMEM((B, _PPW, D), jnp.float32),
            pltpu.VMEM((_PPW, D), jnp.float32),
            pltpu.SemaphoreType.DMA,
            pltpu.SemaphoreType.DMA,
            pltpu.SemaphoreType.DMA,
            pltpu.SemaphoreType.DMA,
            pltpu.SemaphoreType.DMA,
            pltpu.SemaphoreType.DMA,
            pltpu.SemaphoreType.DMA,
        ],
    )
    def sc_embed(idx_hbm, table_hbm, pe_hbm, out_hbm, idx_v, rows_v, pe_v,
                 isem, psem, g0, g1, g2, g3, osem):
        wid = lax.axis_index("s") * _NC + lax.axis_index("c")
        pos0 = wid * _PPW
        gsems = (g0, g1, g2, g3)

        idx_cps = [
            pltpu.async_copy(idx_hbm.at[pl.ds(b * L + pos0, _PPW)],
                             idx_v.at[b], isem)
            for b in range(B)
        ]
        pe_cp = pltpu.async_copy(pe_hbm.at[pl.ds(pos0, _PPW)], pe_v, psem)
        for cp in idx_cps:
            cp.wait()
        gathers = [
            pltpu.async_copy(table_hbm.at[idx_v.at[b]], rows_v.at[b], gsems[b])
            for b in range(B)
        ]
        pe_cp.wait()

        out_cps = []
        for b0 in range(0, B, 2):
            gathers[b0].wait()
            gathers[b0 + 1].wait()

            def body(i, carry, b0=b0):
                for j in range(D // _LANES):
                    sl = pl.ds(j * _LANES, _LANES)
                    p = pe_v[i, sl]
                    for b in (b0, b0 + 1):
                        rows_v[b, i, sl] = rows_v[b, i, sl] * SCALE + p
                return carry

            lax.fori_loop(0, _PPW, body, 0)
            for b in (b0, b0 + 1):
                out_cps.append(pltpu.async_copy(
                    rows_v.at[b], out_hbm.at[pl.ds(b * L + pos0, _PPW)], osem))
        for cp in out_cps:
            cp.wait()

    return sc_embed


_sc_embed = _make_sc_embed()


def kernel(x, table):
    idx = x.reshape(-1)
    pe = jnp.asarray(_PE)
    out = _sc_embed(idx, table, pe)
    return out.reshape(B, L, D)


# speedup vs baseline: 1.2066x; 1.0284x over previous
import functools
import math

import jax
import jax.numpy as jnp
import numpy as np
from jax import lax
from jax.experimental import pallas as pl
from jax.experimental.pallas import tpu as pltpu
from jax.experimental.pallas import tpu_sc as plsc

D = 128
B = 4
L = 2048
BL = B * L
SCALE = math.sqrt(float(D))

_NC = 2
_NS = 16
_NW = _NC * _NS
_PPW = L // _NW
_HALF = _PPW // 2
_LANES = 16


def _pos_encoding(length, depth):
    half = depth // 2
    positions = np.arange(length)[:, None]
    depths = np.arange(half)[None, :] / half
    angle_rates = 1 / 10000 ** depths
    angle_rads = positions * angle_rates
    enc = np.concatenate([np.sin(angle_rads), np.cos(angle_rads)], axis=-1)
    return enc.astype(np.float32)


_PE = _pos_encoding(L, D)


def _make_sc_embed():
    mesh = plsc.VectorSubcoreMesh(
        core_axis_name="c", subcore_axis_name="s", num_cores=_NC
    )

    @functools.partial(
        pl.kernel,
        mesh=mesh,
        out_type=jax.ShapeDtypeStruct((BL, D), jnp.float32),
        scratch_types=[
            pltpu.VMEM((B, _PPW), jnp.int32),
            pltpu.VMEM((B, _PPW, D), jnp.float32),
            pltpu.VMEM((_PPW, D), jnp.float32),
            pltpu.SemaphoreType.DMA,
            pltpu.SemaphoreType.DMA,
            pltpu.SemaphoreType.DMA,
            pltpu.SemaphoreType.DMA,
            pltpu.SemaphoreType.DMA,
            pltpu.SemaphoreType.DMA,
            pltpu.SemaphoreType.DMA,
            pltpu.SemaphoreType.DMA,
            pltpu.SemaphoreType.DMA,
            pltpu.SemaphoreType.DMA,
            pltpu.SemaphoreType.DMA,
        ],
    )
    def sc_embed(idx_hbm, table_hbm, pe_hbm, out_hbm, idx_v, rows_v, pe_v,
                 isem, psem, g00, g01, g10, g11, g20, g21, g30, g31, osem):
        wid = lax.axis_index("s") * _NC + lax.axis_index("c")
        pos0 = wid * _PPW
        gsems = ((g00, g01), (g10, g11), (g20, g21), (g30, g31))

        idx_cps = [
            pltpu.async_copy(idx_hbm.at[pl.ds(b * L + pos0, _PPW)],
                             idx_v.at[b], isem)
            for b in range(B)
        ]
        pe_cp = pltpu.async_copy(pe_hbm.at[pl.ds(pos0, _PPW)], pe_v, psem)
        gathers = {}
        for b in range(B):
            idx_cps[b].wait()
            for h in range(2):
                gathers[(b, h)] = pltpu.async_copy(
                    table_hbm.at[idx_v.at[b, pl.ds(h * _HALF, _HALF)]],
                    rows_v.at[b, pl.ds(h * _HALF, _HALF)],
                    gsems[b][h])
        pe_cp.wait()

        out_cps = []
        for h in range(2):
            for b0 in range(0, B, 2):
                gathers[(b0, h)].wait()
                gathers[(b0 + 1, h)].wait()

                def body(i, carry, b0=b0, h=h):
                    i = i + h * _HALF
                    for j in range(D // _LANES):
                        sl = pl.ds(j * _LANES, _LANES)
                        p = pe_v[i, sl]
                        for b in (b0, b0 + 1):
                            rows_v[b, i, sl] = rows_v[b, i, sl] * SCALE + p
                    return carry

                lax.fori_loop(0, _HALF, body, 0)
                for b in (b0, b0 + 1):
                    out_cps.append(pltpu.async_copy(
                        rows_v.at[b, pl.ds(h * _HALF, _HALF)],
                        out_hbm.at[pl.ds(b * L + pos0 + h * _HALF, _HALF)],
                        osem))
        for cp in out_cps:
            cp.wait()

    return sc_embed


_sc_embed = _make_sc_embed()


def kernel(x, table):
    idx = x.reshape(-1)
    pe = jnp.asarray(_PE)
    out = _sc_embed(idx, table, pe)
    return out.reshape(B, L, D)
